# baseline (speedup 1.0000x reference)
import functools
import jax
import jax.numpy as jnp
from jax import lax
from jax.experimental import pallas as pl
from jax.experimental.pallas import tpu as pltpu
from jax.experimental.pallas import tpu_sc as plsc

B = 16384
D = 64


@jax.jit
def _bprmf_gather(user, pos_item, neg_item, embed_user, embed_item):
    info = plsc.get_sparse_core_info()
    nc, ns = info.num_cores, info.num_subcores
    nw = nc * ns
    bpw = B // nw
    mesh = plsc.VectorSubcoreMesh(core_axis_name="c", subcore_axis_name="s")

    @functools.partial(
        pl.kernel,
        mesh=mesh,
        compiler_params=pltpu.CompilerParams(use_tc_tiling_on_sc=False),
        out_type=(
            jax.ShapeDtypeStruct((B, D), jnp.float32),
            jax.ShapeDtypeStruct((B, D), jnp.float32),
            jax.ShapeDtypeStruct((B, D), jnp.float32),
        ),
        scratch_types=[
            pltpu.VMEM((bpw,), jnp.int32),
            pltpu.VMEM((bpw,), jnp.int32),
            pltpu.VMEM((bpw,), jnp.int32),
            pltpu.VMEM((bpw, D), jnp.float32),
            pltpu.VMEM((bpw, D), jnp.float32),
            pltpu.VMEM((bpw, D), jnp.float32),
            pltpu.SemaphoreType.DMA,
            pltpu.SemaphoreType.DMA,
            pltpu.SemaphoreType.DMA,
        ],
    )
    def k(user_hbm, pos_hbm, neg_hbm, eu_hbm, ei_hbm,
          out_u, out_p, out_n,
          iu, ip, inn, ru, rp, rn, su, sp, sn):
        wid = lax.axis_index("s") * nc + lax.axis_index("c")
        base = wid * bpw
        pltpu.sync_copy(user_hbm.at[pl.ds(base, bpw)], iu)
        pltpu.sync_copy(pos_hbm.at[pl.ds(base, bpw)], ip)
        pltpu.sync_copy(neg_hbm.at[pl.ds(base, bpw)], inn)
        cu = pltpu.async_copy(eu_hbm.at[iu], ru, su)
        cp = pltpu.async_copy(ei_hbm.at[ip], rp, sp)
        cn = pltpu.async_copy(ei_hbm.at[inn], rn, sn)
        cu.wait()
        pltpu.sync_copy(ru, out_u.at[pl.ds(base, bpw)])
        cp.wait()
        pltpu.sync_copy(rp, out_p.at[pl.ds(base, bpw)])
        cn.wait()
        pltpu.sync_copy(rn, out_n.at[pl.ds(base, bpw)])

    return k(user, pos_item, neg_item, embed_user, embed_item)


def kernel(user, pos_item, neg_item, embed_user, embed_item):
    return _bprmf_gather(user, pos_item, neg_item, embed_user, embed_item)
